# baseline (speedup 1.0000x reference)
import dataclasses
import functools

import jax
import jax.numpy as jnp
from jax import lax
from jax.experimental import pallas as pl
from jax.experimental.pallas import tpu as pltpu
from jax.experimental.pallas import tpu_sc as plsc

_N = 4096
_D = 16
_K = 15
_BR = 256
_G = _N // 16
_NW = 32
_RT = _N // _NW
_B = 8
_NB = _RT // _B


def _tc1_body(norm_ref, lat_blk_ref, lat_full_ref, rn_full_ref, dx_ref, mx_ref,
              eye_ref, d12_ref, dz_ref, rank_ref):
    i = pl.program_id(0)
    lat_blk = lat_blk_ref[...]
    lat_full = lat_full_ref[...]
    rn_full = rn_full_ref[...]
    inv_norm = 1.0 / norm_ref[0]
    rn_blk = jnp.sum(lat_blk * lat_blk, axis=1, keepdims=True)

    g = jax.lax.dot_general(lat_blk, lat_full, (((1,), (1,)), ((), ())),
                            preferred_element_type=jnp.float32,
                            precision=jax.lax.Precision.HIGHEST)
    sq = jnp.maximum(rn_blk + rn_full - 2.0 * g, 0.0)
    dz = jnp.sqrt(sq) * inv_norm
    col = jax.lax.broadcasted_iota(jnp.int32, (_BR, _N), 1)
    row = jax.lax.broadcasted_iota(jnp.int32, (_BR, _N), 0) + i * _BR
    dz = jnp.where(col == row, 0.0, dz)
    dz_ref[...] = dz

    dx = dx_ref[...]
    mx = mx_ref[...]
    diff = dx - dz
    d12 = jnp.sum(mx * (diff * diff))

    gt = jax.lax.dot_general(lat_full, lat_blk, (((1,), (1,)), ((), ())),
                             preferred_element_type=jnp.float32)
    rn_full_col = jnp.sum(lat_full * lat_full, axis=1, keepdims=True)
    st = rn_full_col - 2.0 * gt
    colt = jax.lax.broadcasted_iota(jnp.int32, (_N, _BR), 0)
    rowt = jax.lax.broadcasted_iota(jnp.int32, (_N, _BR), 1) + i * _BR
    st = jnp.where(colt == rowt, -1e30, st)

    cand_t = jnp.min(st.reshape(_G, 16, _BR), axis=1)
    cand = jax.lax.dot_general(eye_ref[...], cand_t, (((1,), (1,)), ((), ())),
                               preferred_element_type=jnp.float32,
                               precision=jax.lax.Precision.HIGHEST)
    w = cand
    rank = jnp.zeros((_BR, _G), jnp.float32)
    for t in range(_K + 1):
        mval = jnp.min(w, axis=1, keepdims=True)
        hit = w == mval
        rank = rank + jnp.where(hit, jnp.float32(t + 1), 0.0)
        w = jnp.where(hit, jnp.inf, w)
    rank_ref[...] = rank

    @pl.when(i == 0)
    def _():
        d12_ref[0, 0] = d12

    @pl.when(i != 0)
    def _():
        d12_ref[0, 0] += d12


def _sc_kernel(rank_hbm, dz16_hbm, dx16_hbm, mx16_hbm,
               cdz_hbm, cdx_hbm, cmx_hbm,
               maskv, idxst, gidx0, gidx1,
               gdz0, gdz1, gdx0, gdx1, gmx0, gmx1, sem0, sem1):
    wid = lax.axis_index("s") * plsc.get_sparse_core_info().num_cores \
        + lax.axis_index("c")
    tile_base = wid * _RT
    pltpu.sync_copy(rank_hbm.at[pl.ds(tile_base, _RT)], maskv)

    ids = [lax.iota(jnp.int32, 16) + c * 16 for c in range(16)]

    def build_batch(b, gidx):
        for rloc in range(_B):
            r = tile_base + b * _B + rloc
            for c in range(16):
                ranks = maskv[r - tile_base, pl.ds(c * 16, 16)]
                m = ranks > 0.5
                slots = lax.convert_element_type(ranks, jnp.int32) - 1
                plsc.store_scatter(idxst, [slots], ids[c], mask=m)
            giv = idxst[...]
            gidx[pl.ds(rloc * 16, 16)] = giv + r * _G

    def issue(b, gidx, gdz, gdx, gmx, sem):
        build_batch(b, gidx)
        pltpu.async_copy(dz16_hbm.at[gidx], gdz, sem)
        pltpu.async_copy(dx16_hbm.at[gidx], gdx, sem)
        pltpu.async_copy(mx16_hbm.at[gidx], gmx, sem)

    def drain_write(b, gdz, gdx, gmx, sem):
        base = (tile_base + b * _B) * 16
        pltpu.make_async_copy(dz16_hbm.at[pl.ds(0, _B * 16)], gdz, sem).wait()
        pltpu.make_async_copy(dz16_hbm.at[pl.ds(0, _B * 16)], gdx, sem).wait()
        pltpu.make_async_copy(dz16_hbm.at[pl.ds(0, _B * 16)], gmx, sem).wait()
        pltpu.sync_copy(gdz, cdz_hbm.at[pl.ds(base, _B * 16)])
        pltpu.sync_copy(gdx, cdx_hbm.at[pl.ds(base, _B * 16)])
        pltpu.sync_copy(gmx, cmx_hbm.at[pl.ds(base, _B * 16)])

    issue(jnp.int32(0), gidx0, gdz0, gdx0, gmx0, sem0)

    @pl.loop(0, (_NB - 2) // 2)
    def _(j):
        b = 2 * j
        issue(b + 1, gidx1, gdz1, gdx1, gmx1, sem1)
        drain_write(b, gdz0, gdx0, gmx0, sem0)
        issue(b + 2, gidx0, gdz0, gdx0, gmx0, sem0)
        drain_write(b + 1, gdz1, gdx1, gmx1, sem1)

    last = jnp.int32(_NB - 1)
    issue(last, gidx1, gdz1, gdx1, gmx1, sem1)
    drain_write(last - 1, gdz0, gdx0, gmx0, sem0)
    drain_write(last, gdz1, gdx1, gmx1, sem1)


def _tc2_body(cdz_ref, cdx_ref, cmx_ref, d21_ref, ov_ref):
    i = pl.program_id(0)
    cdz = cdz_ref[...]
    w = cdz
    for _ in range(_K + 1):
        mval = jnp.min(w, axis=1, keepdims=True)
        w = jnp.where(w == mval, jnp.inf, w)
    mask_z = jnp.where(jnp.isinf(w) & (cdz > 0.0), 1.0, 0.0)
    diff = cdx_ref[...] - cdz
    d21 = jnp.sum(mask_z * (diff * diff))
    ov = jnp.sum(mask_z * cmx_ref[...])

    @pl.when(i == 0)
    def _():
        d21_ref[0, 0] = d21
        ov_ref[0, 0] = ov

    @pl.when(i != 0)
    def _():
        d21_ref[0, 0] += d21
        ov_ref[0, 0] += ov


@jax.jit
def kernel(latent, latent_norm, dist_X, pair_mask_X):
    n, k = _N, _K
    rn_full = jnp.sum(latent * latent, axis=1)[None, :]
    norm = latent_norm.reshape((1,))
    eye = jnp.eye(_BR, dtype=jnp.float32)
    scalar_spec = pl.BlockSpec(memory_space=pltpu.SMEM)
    d12o, dz, rankm = pl.pallas_call(
        _tc1_body,
        grid=(n // _BR,),
        in_specs=[
            scalar_spec,
            pl.BlockSpec((_BR, _D), lambda i: (i, 0)),
            pl.BlockSpec((_N, _D), lambda i: (0, 0)),
            pl.BlockSpec((1, _N), lambda i: (0, 0)),
            pl.BlockSpec((_BR, _N), lambda i: (i, 0)),
            pl.BlockSpec((_BR, _N), lambda i: (i, 0)),
            pl.BlockSpec((_BR, _BR), lambda i: (0, 0)),
        ],
        out_specs=[
            pl.BlockSpec((1, 1), lambda i: (0, 0), memory_space=pltpu.SMEM),
            pl.BlockSpec((_BR, _N), lambda i: (i, 0)),
            pl.BlockSpec((_BR, _G), lambda i: (i, 0)),
        ],
        out_shape=[
            jax.ShapeDtypeStruct((1, 1), jnp.float32),
            jax.ShapeDtypeStruct((n, n), jnp.float32),
            jax.ShapeDtypeStruct((n, _G), jnp.float32),
        ],
    )(norm, latent, latent, rn_full, dist_X, pair_mask_X, eye)

    mesh = plsc.VectorSubcoreMesh(core_axis_name="c", subcore_axis_name="s")
    cp = pltpu.CompilerParams(needs_layout_passes=False,
                              use_tc_tiling_on_sc=False)
    sc = pl.kernel(
        _sc_kernel,
        out_type=[jax.ShapeDtypeStruct((n * 16, 16), jnp.float32)] * 3,
        mesh=mesh,
        compiler_params=cp,
        scratch_types=[
            pltpu.VMEM((_RT, _G), jnp.float32),
            pltpu.VMEM((16,), jnp.int32),
            pltpu.VMEM((_B * 16,), jnp.int32),
            pltpu.VMEM((_B * 16,), jnp.int32),
            pltpu.VMEM((_B * 16, 16), jnp.float32),
            pltpu.VMEM((_B * 16, 16), jnp.float32),
            pltpu.VMEM((_B * 16, 16), jnp.float32),
            pltpu.VMEM((_B * 16, 16), jnp.float32),
            pltpu.VMEM((_B * 16, 16), jnp.float32),
            pltpu.VMEM((_B * 16, 16), jnp.float32),
            pltpu.SemaphoreType.DMA,
            pltpu.SemaphoreType.DMA,
        ],
    )
    cdz, cdx, cmx = sc(rankm,
                       dz.reshape(n * _G, 16),
                       dist_X.reshape(n * _G, 16),
                       pair_mask_X.reshape(n * _G, 16))

    _BR2 = 1024
    d21o, ovo = pl.pallas_call(
        _tc2_body,
        grid=(n // _BR2,),
        in_specs=[
            pl.BlockSpec((_BR2, 256), lambda i: (i, 0)),
            pl.BlockSpec((_BR2, 256), lambda i: (i, 0)),
            pl.BlockSpec((_BR2, 256), lambda i: (i, 0)),
        ],
        out_specs=[
            pl.BlockSpec((1, 1), lambda i: (0, 0), memory_space=pltpu.SMEM),
            pl.BlockSpec((1, 1), lambda i: (0, 0), memory_space=pltpu.SMEM),
        ],
        out_shape=[
            jax.ShapeDtypeStruct((1, 1), jnp.float32),
            jax.ShapeDtypeStruct((1, 1), jnp.float32),
        ],
    )(cdz.reshape(n, 256), cdx.reshape(n, 256), cmx.reshape(n, 256))

    d12 = d12o[0, 0]
    d21 = d21o[0, 0]
    ov = ovo[0, 0]
    distance = d12 + d21
    matched_pairs = ov / (n * k)
    return (distance, matched_pairs, d12, d21)


# speedup vs baseline: 1.1468x; 1.1468x over previous
import dataclasses
import functools

import jax
import jax.numpy as jnp
from jax import lax
from jax.experimental import pallas as pl
from jax.experimental.pallas import tpu as pltpu
from jax.experimental.pallas import tpu_sc as plsc

_N = 4096
_D = 16
_K = 15
_BR = 256
_G = _N // 16
_NW = 32
_RT = _N // _NW
_B = 8
_NB = _RT // _B


def _tc1_body(norm_ref, lat_blk_ref, lat_full_ref, rn_full_ref, dx_ref, mx_ref,
              eye_ref, d12_ref, dz_ref, tp_ref, rank_ref):
    i = pl.program_id(0)
    lat_blk = lat_blk_ref[...]
    lat_full = lat_full_ref[...]
    rn_full = rn_full_ref[...]
    inv_norm = 1.0 / norm_ref[0]
    rn_blk = jnp.sum(lat_blk * lat_blk, axis=1, keepdims=True)

    g = jax.lax.dot_general(lat_blk, lat_full, (((1,), (1,)), ((), ())),
                            preferred_element_type=jnp.float32,
                            precision=jax.lax.Precision.HIGHEST)
    sq = jnp.maximum(rn_blk + rn_full - 2.0 * g, 0.0)
    dz = jnp.sqrt(sq) * inv_norm
    col = jax.lax.broadcasted_iota(jnp.int32, (_BR, _N), 1)
    row = jax.lax.broadcasted_iota(jnp.int32, (_BR, _N), 0) + i * _BR
    dz = jnp.where(col == row, 0.0, dz)
    dz_ref[...] = dz

    dx = dx_ref[...]
    mx = mx_ref[...]
    diff = dx - dz
    dsq = diff * diff
    d12 = jnp.sum(mx * dsq)
    tp_ref[...] = jnp.where(mx > 0.5, -dsq, dsq)

    gt = jax.lax.dot_general(lat_full, lat_blk, (((1,), (1,)), ((), ())),
                             preferred_element_type=jnp.float32)
    rn_full_col = jnp.sum(lat_full * lat_full, axis=1, keepdims=True)
    st = rn_full_col - 2.0 * gt
    colt = jax.lax.broadcasted_iota(jnp.int32, (_N, _BR), 0)
    rowt = jax.lax.broadcasted_iota(jnp.int32, (_N, _BR), 1) + i * _BR
    st = jnp.where(colt == rowt, -1e30, st)

    cand_t = jnp.min(st.reshape(_G, 16, _BR), axis=1)
    cand = jax.lax.dot_general(eye_ref[...], cand_t, (((1,), (1,)), ((), ())),
                               preferred_element_type=jnp.float32,
                               precision=jax.lax.Precision.HIGHEST)
    w = cand
    rank = jnp.zeros((_BR, _G), jnp.float32)
    for t in range(_K + 1):
        mval = jnp.min(w, axis=1, keepdims=True)
        hit = w == mval
        rank = rank + jnp.where(hit, jnp.float32(t + 1), 0.0)
        w = jnp.where(hit, jnp.inf, w)
    rank_ref[...] = rank

    @pl.when(i == 0)
    def _():
        d12_ref[0, 0] = d12

    @pl.when(i != 0)
    def _():
        d12_ref[0, 0] += d12


def _sc_kernel(rank_hbm, dz16_hbm, tp16_hbm,
               cdz_hbm, ctp_hbm,
               maskv, idxst, gidx0, gidx1,
               gdz0, gdz1, gtp0, gtp1, sem0, sem1):
    wid = lax.axis_index("s") * plsc.get_sparse_core_info().num_cores \
        + lax.axis_index("c")
    tile_base = wid * _RT
    pltpu.sync_copy(rank_hbm.at[pl.ds(tile_base, _RT)], maskv)

    ids = [lax.iota(jnp.int32, 16) + c * 16 for c in range(16)]

    def build_batch(b, gidx):
        for rloc in range(_B):
            r = tile_base + b * _B + rloc
            for c in range(16):
                ranks = maskv[r - tile_base, pl.ds(c * 16, 16)]
                m = ranks > 0.5
                slots = lax.convert_element_type(ranks, jnp.int32) - 1
                plsc.store_scatter(idxst, [slots], ids[c], mask=m)
            giv = idxst[...]
            gidx[pl.ds(rloc * 16, 16)] = giv + r * _G

    def issue(b, gidx, gdz, gtp, sem):
        build_batch(b, gidx)
        pltpu.async_copy(dz16_hbm.at[gidx], gdz, sem)
        pltpu.async_copy(tp16_hbm.at[gidx], gtp, sem)

    def drain_write(b, gdz, gtp, sem):
        base = (tile_base + b * _B) * 16
        pltpu.make_async_copy(dz16_hbm.at[pl.ds(0, _B * 16)], gdz, sem).wait()
        pltpu.make_async_copy(dz16_hbm.at[pl.ds(0, _B * 16)], gtp, sem).wait()
        pltpu.sync_copy(gdz, cdz_hbm.at[pl.ds(base, _B * 16)])
        pltpu.sync_copy(gtp, ctp_hbm.at[pl.ds(base, _B * 16)])

    issue(jnp.int32(0), gidx0, gdz0, gtp0, sem0)

    @pl.loop(0, (_NB - 2) // 2)
    def _(j):
        b = 2 * j
        issue(b + 1, gidx1, gdz1, gtp1, sem1)
        drain_write(b, gdz0, gtp0, sem0)
        issue(b + 2, gidx0, gdz0, gtp0, sem0)
        drain_write(b + 1, gdz1, gtp1, sem1)

    last = jnp.int32(_NB - 1)
    issue(last, gidx1, gdz1, gtp1, sem1)
    drain_write(last - 1, gdz0, gtp0, sem0)
    drain_write(last, gdz1, gtp1, sem1)


def _tc2_body(cdz_ref, ctp_ref, d21_ref, ov_ref):
    i = pl.program_id(0)
    cdz = cdz_ref[...]
    w = cdz
    for _ in range(_K + 1):
        mval = jnp.min(w, axis=1, keepdims=True)
        w = jnp.where(w == mval, jnp.inf, w)
    mask_z = jnp.where(jnp.isinf(w) & (cdz > 0.0), 1.0, 0.0)
    ctp = ctp_ref[...]
    bits = jax.lax.bitcast_convert_type(ctp, jnp.int32)
    mxv = jnp.where(bits < 0, 1.0, 0.0)
    d21 = jnp.sum(mask_z * jnp.abs(ctp))
    ov = jnp.sum(mask_z * mxv)

    @pl.when(i == 0)
    def _():
        d21_ref[0, 0] = d21
        ov_ref[0, 0] = ov

    @pl.when(i != 0)
    def _():
        d21_ref[0, 0] += d21
        ov_ref[0, 0] += ov


@jax.jit
def kernel(latent, latent_norm, dist_X, pair_mask_X):
    n, k = _N, _K
    rn_full = jnp.sum(latent * latent, axis=1)[None, :]
    norm = latent_norm.reshape((1,))
    eye = jnp.eye(_BR, dtype=jnp.float32)
    scalar_spec = pl.BlockSpec(memory_space=pltpu.SMEM)
    d12o, dz, tpk, rankm = pl.pallas_call(
        _tc1_body,
        grid=(n // _BR,),
        in_specs=[
            scalar_spec,
            pl.BlockSpec((_BR, _D), lambda i: (i, 0)),
            pl.BlockSpec((_N, _D), lambda i: (0, 0)),
            pl.BlockSpec((1, _N), lambda i: (0, 0)),
            pl.BlockSpec((_BR, _N), lambda i: (i, 0)),
            pl.BlockSpec((_BR, _N), lambda i: (i, 0)),
            pl.BlockSpec((_BR, _BR), lambda i: (0, 0)),
        ],
        out_specs=[
            pl.BlockSpec((1, 1), lambda i: (0, 0), memory_space=pltpu.SMEM),
            pl.BlockSpec((_BR, _N), lambda i: (i, 0)),
            pl.BlockSpec((_BR, _N), lambda i: (i, 0)),
            pl.BlockSpec((_BR, _G), lambda i: (i, 0)),
        ],
        out_shape=[
            jax.ShapeDtypeStruct((1, 1), jnp.float32),
            jax.ShapeDtypeStruct((n, n), jnp.float32),
            jax.ShapeDtypeStruct((n, n), jnp.float32),
            jax.ShapeDtypeStruct((n, _G), jnp.float32),
        ],
    )(norm, latent, latent, rn_full, dist_X, pair_mask_X, eye)

    mesh = plsc.VectorSubcoreMesh(core_axis_name="c", subcore_axis_name="s")
    cp = pltpu.CompilerParams(needs_layout_passes=False,
                              use_tc_tiling_on_sc=False)
    sc = pl.kernel(
        _sc_kernel,
        out_type=[jax.ShapeDtypeStruct((n * 16, 16), jnp.float32)] * 2,
        mesh=mesh,
        compiler_params=cp,
        scratch_types=[
            pltpu.VMEM((_RT, _G), jnp.float32),
            pltpu.VMEM((16,), jnp.int32),
            pltpu.VMEM((_B * 16,), jnp.int32),
            pltpu.VMEM((_B * 16,), jnp.int32),
            pltpu.VMEM((_B * 16, 16), jnp.float32),
            pltpu.VMEM((_B * 16, 16), jnp.float32),
            pltpu.VMEM((_B * 16, 16), jnp.float32),
            pltpu.VMEM((_B * 16, 16), jnp.float32),
            pltpu.SemaphoreType.DMA,
            pltpu.SemaphoreType.DMA,
        ],
    )
    cdz, ctp = sc(rankm,
                  dz.reshape(n * _G, 16),
                  tpk.reshape(n * _G, 16))

    _BR2 = 1024
    d21o, ovo = pl.pallas_call(
        _tc2_body,
        grid=(n // _BR2,),
        in_specs=[
            pl.BlockSpec((_BR2, 256), lambda i: (i, 0)),
            pl.BlockSpec((_BR2, 256), lambda i: (i, 0)),
        ],
        out_specs=[
            pl.BlockSpec((1, 1), lambda i: (0, 0), memory_space=pltpu.SMEM),
            pl.BlockSpec((1, 1), lambda i: (0, 0), memory_space=pltpu.SMEM),
        ],
        out_shape=[
            jax.ShapeDtypeStruct((1, 1), jnp.float32),
            jax.ShapeDtypeStruct((1, 1), jnp.float32),
        ],
    )(cdz.reshape(n, 256), ctp.reshape(n, 256))

    d12 = d12o[0, 0]
    d21 = d21o[0, 0]
    ov = ovo[0, 0]
    distance = d12 + d21
    matched_pairs = ov / (n * k)
    return (distance, matched_pairs, d12, d21)


# speedup vs baseline: 1.1548x; 1.0070x over previous
import functools

import jax
import jax.numpy as jnp
from jax import lax
from jax.experimental import pallas as pl
from jax.experimental.pallas import tpu as pltpu
from jax.experimental.pallas import tpu_sc as plsc

_N = 4096
_D = 16
_K = 15
_BR = 256
_G = _N // 16
_NH = _N // 2
_NW = 32
_RT = _NH // _NW
_B = 8
_NB = _RT // _B
_BR2 = 1024


def _tc1_body(roff, norm_ref, lat_blk_ref, lat_full_ref, rn_full_ref, dx_ref,
              mx_ref, eye_ref, d12_ref, dz_ref, tp_ref, rank_ref):
    i = pl.program_id(0)
    lat_blk = lat_blk_ref[...]
    lat_full = lat_full_ref[...]
    rn_full = rn_full_ref[...]
    inv_norm = 1.0 / norm_ref[0]
    rn_blk = jnp.sum(lat_blk * lat_blk, axis=1, keepdims=True)

    g = jax.lax.dot_general(lat_blk, lat_full, (((1,), (1,)), ((), ())),
                            preferred_element_type=jnp.float32,
                            precision=jax.lax.Precision.HIGHEST)
    sq = jnp.maximum(rn_blk + rn_full - 2.0 * g, 0.0)
    dz = jnp.sqrt(sq) * inv_norm
    col = jax.lax.broadcasted_iota(jnp.int32, (_BR, _N), 1)
    row = jax.lax.broadcasted_iota(jnp.int32, (_BR, _N), 0) + (i * _BR + roff)
    dz = jnp.where(col == row, 0.0, dz)
    dz_ref[...] = dz

    dx = dx_ref[...]
    mx = mx_ref[...]
    diff = dx - dz
    dsq = diff * diff
    d12 = jnp.sum(mx * dsq)
    tp_ref[...] = jnp.where(mx > 0.5, -dsq, dsq)

    gt = jax.lax.dot_general(lat_full, lat_blk, (((1,), (1,)), ((), ())),
                             preferred_element_type=jnp.float32)
    rn_full_col = jnp.sum(lat_full * lat_full, axis=1, keepdims=True)
    st = rn_full_col - 2.0 * gt
    colt = jax.lax.broadcasted_iota(jnp.int32, (_N, _BR), 0)
    rowt = jax.lax.broadcasted_iota(jnp.int32, (_N, _BR), 1) + (i * _BR + roff)
    st = jnp.where(colt == rowt, -1e30, st)

    cand_t = jnp.min(st.reshape(_G, 16, _BR), axis=1)
    cand = jax.lax.dot_general(eye_ref[...], cand_t, (((1,), (1,)), ((), ())),
                               preferred_element_type=jnp.float32,
                               precision=jax.lax.Precision.HIGHEST)
    w = cand
    rank = jnp.zeros((_BR, _G), jnp.float32)
    for t in range(_K + 1):
        mval = jnp.min(w, axis=1, keepdims=True)
        hit = w == mval
        rank = rank + jnp.where(hit, jnp.float32(t + 1), 0.0)
        w = jnp.where(hit, jnp.inf, w)
    rank_ref[...] = rank

    @pl.when(i == 0)
    def _():
        d12_ref[0, 0] = d12

    @pl.when(i != 0)
    def _():
        d12_ref[0, 0] += d12


def _sc_kernel(rank_hbm, dz16_hbm, tp16_hbm,
               cdz_hbm, ctp_hbm,
               maskv, idxst, gidx0, gidx1,
               gdz0, gdz1, gtp0, gtp1, sem0, sem1):
    wid = lax.axis_index("s") * plsc.get_sparse_core_info().num_cores \
        + lax.axis_index("c")
    tile_base = wid * _RT
    pltpu.sync_copy(rank_hbm.at[pl.ds(tile_base, _RT)], maskv)

    ids = [lax.iota(jnp.int32, 16) + c * 16 for c in range(16)]

    def build_batch(b, gidx):
        for rloc in range(_B):
            r = tile_base + b * _B + rloc
            for c in range(16):
                ranks = maskv[r - tile_base, pl.ds(c * 16, 16)]
                m = ranks > 0.5
                slots = lax.convert_element_type(ranks, jnp.int32) - 1
                plsc.store_scatter(idxst, [slots], ids[c], mask=m)
            giv = idxst[...]
            gidx[pl.ds(rloc * 16, 16)] = giv + r * _G

    def issue(b, gidx, gdz, gtp, sem):
        build_batch(b, gidx)
        pltpu.async_copy(dz16_hbm.at[gidx], gdz, sem)
        pltpu.async_copy(tp16_hbm.at[gidx], gtp, sem)

    def drain_write(b, gdz, gtp, sem):
        base = (tile_base + b * _B) * 16
        pltpu.make_async_copy(dz16_hbm.at[pl.ds(0, _B * 16)], gdz, sem).wait()
        pltpu.make_async_copy(dz16_hbm.at[pl.ds(0, _B * 16)], gtp, sem).wait()
        pltpu.sync_copy(gdz, cdz_hbm.at[pl.ds(base, _B * 16)])
        pltpu.sync_copy(gtp, ctp_hbm.at[pl.ds(base, _B * 16)])

    issue(jnp.int32(0), gidx0, gdz0, gtp0, sem0)

    @pl.loop(0, (_NB - 2) // 2)
    def _(j):
        b = 2 * j
        issue(b + 1, gidx1, gdz1, gtp1, sem1)
        drain_write(b, gdz0, gtp0, sem0)
        issue(b + 2, gidx0, gdz0, gtp0, sem0)
        drain_write(b + 1, gdz1, gtp1, sem1)

    last = jnp.int32(_NB - 1)
    issue(last, gidx1, gdz1, gtp1, sem1)
    drain_write(last - 1, gdz0, gtp0, sem0)
    drain_write(last, gdz1, gtp1, sem1)


def _tc2_body(cdz_ref, ctp_ref, d21_ref, ov_ref):
    i = pl.program_id(0)
    cdz = cdz_ref[...]
    w = cdz
    for _ in range(_K + 1):
        mval = jnp.min(w, axis=1, keepdims=True)
        w = jnp.where(w == mval, jnp.inf, w)
    mask_z = jnp.where(jnp.isinf(w) & (cdz > 0.0), 1.0, 0.0)
    ctp = ctp_ref[...]
    bits = jax.lax.bitcast_convert_type(ctp, jnp.int32)
    mxv = jnp.where(bits < 0, 1.0, 0.0)
    d21 = jnp.sum(mask_z * jnp.abs(ctp))
    ov = jnp.sum(mask_z * mxv)

    @pl.when(i == 0)
    def _():
        d21_ref[0, 0] = d21
        ov_ref[0, 0] = ov

    @pl.when(i != 0)
    def _():
        d21_ref[0, 0] += d21
        ov_ref[0, 0] += ov


def _half_pipeline(h, norm, latent, rn_full, dist_X, pair_mask_X, eye):
    n = _N
    nblk = _NH // _BR
    boff = h * nblk
    scalar_spec = pl.BlockSpec(memory_space=pltpu.SMEM)
    d12o, dz, tpk, rankm = pl.pallas_call(
        functools.partial(_tc1_body, h * _NH),
        grid=(nblk,),
        in_specs=[
            scalar_spec,
            pl.BlockSpec((_BR, _D), lambda i: (i + boff, 0)),
            pl.BlockSpec((_N, _D), lambda i: (0, 0)),
            pl.BlockSpec((1, _N), lambda i: (0, 0)),
            pl.BlockSpec((_BR, _N), lambda i: (i + boff, 0)),
            pl.BlockSpec((_BR, _N), lambda i: (i + boff, 0)),
            pl.BlockSpec((_BR, _BR), lambda i: (0, 0)),
        ],
        out_specs=[
            pl.BlockSpec((1, 1), lambda i: (0, 0), memory_space=pltpu.SMEM),
            pl.BlockSpec((_BR, _N), lambda i: (i, 0)),
            pl.BlockSpec((_BR, _N), lambda i: (i, 0)),
            pl.BlockSpec((_BR, _G), lambda i: (i, 0)),
        ],
        out_shape=[
            jax.ShapeDtypeStruct((1, 1), jnp.float32),
            jax.ShapeDtypeStruct((_NH, n), jnp.float32),
            jax.ShapeDtypeStruct((_NH, n), jnp.float32),
            jax.ShapeDtypeStruct((_NH, _G), jnp.float32),
        ],
    )(norm, latent, latent, rn_full, dist_X, pair_mask_X, eye)

    mesh = plsc.VectorSubcoreMesh(core_axis_name="c", subcore_axis_name="s")
    cp = pltpu.CompilerParams(needs_layout_passes=False,
                              use_tc_tiling_on_sc=False)
    sc = pl.kernel(
        _sc_kernel,
        out_type=[jax.ShapeDtypeStruct((_NH * 16, 16), jnp.float32)] * 2,
        mesh=mesh,
        compiler_params=cp,
        scratch_types=[
            pltpu.VMEM((_RT, _G), jnp.float32),
            pltpu.VMEM((16,), jnp.int32),
            pltpu.VMEM((_B * 16,), jnp.int32),
            pltpu.VMEM((_B * 16,), jnp.int32),
            pltpu.VMEM((_B * 16, 16), jnp.float32),
            pltpu.VMEM((_B * 16, 16), jnp.float32),
            pltpu.VMEM((_B * 16, 16), jnp.float32),
            pltpu.VMEM((_B * 16, 16), jnp.float32),
            pltpu.SemaphoreType.DMA,
            pltpu.SemaphoreType.DMA,
        ],
    )
    cdz, ctp = sc(rankm,
                  dz.reshape(_NH * _G, 16),
                  tpk.reshape(_NH * _G, 16))

    d21o, ovo = pl.pallas_call(
        _tc2_body,
        grid=(_NH // _BR2,),
        in_specs=[
            pl.BlockSpec((_BR2, 256), lambda i: (i, 0)),
            pl.BlockSpec((_BR2, 256), lambda i: (i, 0)),
        ],
        out_specs=[
            pl.BlockSpec((1, 1), lambda i: (0, 0), memory_space=pltpu.SMEM),
            pl.BlockSpec((1, 1), lambda i: (0, 0), memory_space=pltpu.SMEM),
        ],
        out_shape=[
            jax.ShapeDtypeStruct((1, 1), jnp.float32),
            jax.ShapeDtypeStruct((1, 1), jnp.float32),
        ],
    )(cdz.reshape(_NH, 256), ctp.reshape(_NH, 256))
    return d12o[0, 0], d21o[0, 0], ovo[0, 0]


@jax.jit
def kernel(latent, latent_norm, dist_X, pair_mask_X):
    n, k = _N, _K
    rn_full = jnp.sum(latent * latent, axis=1)[None, :]
    norm = latent_norm.reshape((1,))
    eye = jnp.eye(_BR, dtype=jnp.float32)
    d12a, d21a, ova = _half_pipeline(0, norm, latent, rn_full, dist_X,
                                     pair_mask_X, eye)
    d12b, d21b, ovb = _half_pipeline(1, norm, latent, rn_full, dist_X,
                                     pair_mask_X, eye)
    d12 = d12a + d12b
    d21 = d21a + d21b
    ov = ova + ovb
    distance = d12 + d21
    matched_pairs = ov / (n * k)
    return (distance, matched_pairs, d12, d21)
